# baseline (speedup 1.0000x reference)
import functools

import jax
import jax.numpy as jnp
from jax import lax
from jax.experimental import pallas as pl
from jax.experimental.pallas import tpu as pltpu
from jax.experimental.pallas import tpu_sc as plsc

DIM = 32
LANES = 16
MARGIN = 1.0
EPS = 1e-6

NC = 2
NS = 16
DPC = DIM // NC

SB = 32


@functools.lru_cache(maxsize=None)
def _make_sc_kernel(nent, tpw):
    mesh = plsc.VectorSubcoreMesh(core_axis_name="c", subcore_axis_name="s")
    b = NS * tpw
    npair = tpw // (2 * SB)
    w = 6 * SB
    wpad = ((w + 127) // 128) * 128

    @functools.partial(
        pl.kernel,
        mesh=mesh,
        out_type=jax.ShapeDtypeStruct((NC, 2, b), jnp.float32),
        scratch_types=[
            pltpu.VMEM_SHARED((nent,), jnp.float32),
            pltpu.VMEM_SHARED((nent,), jnp.float32),
            pltpu.VMEM((wpad,), jnp.int32),
            pltpu.VMEM((wpad,), jnp.int32),
            pltpu.VMEM((w,), jnp.float32),
            pltpu.VMEM((w,), jnp.float32),
            pltpu.VMEM((tpw,), jnp.float32),
            pltpu.VMEM((tpw,), jnp.float32),
            pltpu.SemaphoreType.DMA,
            pltpu.SemaphoreType.DMA,
            pltpu.SemaphoreType.DMA,
        ],
    )
    def sc_kernel(idx_hbm, ent_t, rel_t, out_hbm,
                  plane_e, plane_r, idx0, idx1, dat0, dat1,
                  psum, nsum, sem_p, sem_i, sem_g):
        ibufs = (idx0, idx1)
        dbufs = (dat0, dat1)

        c = lax.axis_index("c")
        s = lax.axis_index("s")

        def issue_planes(k):
            h0 = None
            for cc in range(NC):
                @pl.when((c == cc) & (s == 0))
                def _():
                    nonlocal h0
                    h = pltpu.async_copy(ent_t.at[cc * DPC + k],
                                         plane_e, sem_p)
                    h0 = h if cc == 0 else h0
                @pl.when((c == cc) & (s == 1))
                def _():
                    pltpu.async_copy(rel_t.at[cc * DPC + k], plane_r, sem_p)
            return h0

        def wait_planes(handle):
            @pl.when((s == 0) | (s == 1))
            def _():
                handle.wait()

        def issue_idx(set_, sb):
            start = pl.multiple_of(sb * wpad, wpad)
            return pltpu.async_copy(idx_hbm.at[s, pl.ds(start, wpad)],
                                    ibufs[set_], sem_i)

        def drain_idx(set_):
            pltpu.make_async_copy(idx_hbm.at[s, pl.ds(0, wpad)],
                                  ibufs[set_], sem_i).wait()

        def issue_gathers(iset, dset):
            ib = ibufs[iset]
            db = dbufs[dset]
            return [
                pltpu.async_copy(
                    plane_e.at[ib.at[pl.ds(0, 4 * SB)]],
                    db.at[pl.ds(0, 4 * SB)], sem_g),
                pltpu.async_copy(
                    plane_r.at[ib.at[pl.ds(4 * SB, 2 * SB)]],
                    db.at[pl.ds(4 * SB, 2 * SB)], sem_g),
            ]

        def wait_gathers(hs):
            for h in hs:
                h.wait()

        def fold(dset, sb, k):
            db = dbufs[dset]
            base = sb * SB

            def grp(g, _):
                def role(r):
                    return db[pl.ds(
                        pl.multiple_of(r * SB + g * LANES, LANES), LANES)]
                osl = pl.ds(pl.multiple_of(base + g * LANES, LANES), LANES)
                dp = role(0) + role(4) - role(1) + jnp.float32(EPS)
                dn = role(2) + role(5) - role(3) + jnp.float32(EPS)
                if k == 0:
                    psum[osl] = dp * dp
                    nsum[osl] = dn * dn
                else:
                    psum[osl] = psum[osl] + dp * dp
                    nsum[osl] = nsum[osl] + dn * dn
                return 0

            lax.fori_loop(0, SB // LANES, grp, 0)

        def drain_gathers(dset):
            db = dbufs[dset]
            pltpu.make_async_copy(
                plane_e.at[ibufs[0].at[pl.ds(0, 4 * SB)]],
                db.at[pl.ds(0, 4 * SB)], sem_g).wait()
            pltpu.make_async_copy(
                plane_r.at[ibufs[0].at[pl.ds(4 * SB, 2 * SB)]],
                db.at[pl.ds(4 * SB, 2 * SB)], sem_g).wait()

        hplane = issue_planes(0)
        issue_idx(0, 0)
        issue_idx(1, 1)

        for k in range(DPC):
            wait_planes(hplane)
            plsc.subcore_barrier()

            drain_idx(0)
            issue_gathers(0, 0)

            def pair(j, _):
                drain_idx(1)
                issue_gathers(1, 1)
                issue_idx(0, 2 * j + 2)
                drain_gathers(0)
                fold(0, 2 * j, k)
                drain_idx(0)
                issue_gathers(0, 0)
                issue_idx(1, 2 * j + 3)
                drain_gathers(1)
                fold(1, 2 * j + 1, k)
                return 0

            lax.fori_loop(0, npair - 1, pair, 0)

            drain_idx(1)
            issue_gathers(1, 1)
            issue_idx(0, 0)
            drain_gathers(0)
            fold(0, 2 * npair - 2, k)
            issue_idx(1, 1)
            drain_gathers(1)
            fold(1, 2 * npair - 1, k)

            plsc.subcore_barrier()
            if k + 1 < DPC:
                hplane = issue_planes(k + 1)

        drain_idx(0)
        drain_idx(1)

        pltpu.sync_copy(psum, out_hbm.at[c, 0, pl.ds(s * tpw, tpw)])
        pltpu.sync_copy(nsum, out_hbm.at[c, 1, pl.ds(s * tpw, tpw)])

    return sc_kernel


def _finish_body(inv_b, parts_ref, o_ref):
    p = parts_ref[0, 0, :] + parts_ref[1, 0, :]
    n = parts_ref[0, 1, :] + parts_ref[1, 1, :]
    hinge = jnp.maximum(jnp.sqrt(p) - jnp.sqrt(n) + jnp.float32(MARGIN),
                        jnp.float32(0.0))
    o_ref[0, 0] = jnp.sum(hinge) * jnp.float32(inv_b)


@functools.lru_cache(maxsize=None)
def _make_finish(b):
    return pl.pallas_call(
        functools.partial(_finish_body, 1.0 / b),
        out_shape=jax.ShapeDtypeStruct((1, 1), jnp.float32),
        out_specs=pl.BlockSpec(memory_space=pltpu.SMEM),
    )


def kernel(posX, negX, alpha, beta, entityEmbed, relationEmbed):
    b = posX.shape[0]
    tpw = b // NS
    nsub = tpw // SB
    nent = entityEmbed.shape[0]
    w = 6 * SB
    wpad = ((w + 127) // 128) * 128
    x = jnp.concatenate([posX, negX], axis=1)
    xp = x[:, jnp.array([0, 2, 3, 5, 1, 4])]
    idx = (xp.reshape(NS, nsub, SB, 6)
             .transpose(0, 1, 3, 2)
             .reshape(NS, nsub, w))
    idx = jnp.pad(idx, ((0, 0), (0, 0), (0, wpad - w)))
    idx = idx.reshape(NS, nsub * wpad)
    parts = _make_sc_kernel(nent, tpw)(
        idx, entityEmbed.T, relationEmbed.T)
    return _make_finish(b)(parts)[0, 0]


# speedup vs baseline: 1.3674x; 1.3674x over previous
import functools

import jax
import jax.numpy as jnp
from jax import lax
from jax.experimental import pallas as pl
from jax.experimental.pallas import tpu as pltpu
from jax.experimental.pallas import tpu_sc as plsc

DIM = 32
LANES = 16
MARGIN = 1.0
EPS = 1e-6

NC = 2
NS = 16
DPC = DIM // NC

SB = 64


@functools.lru_cache(maxsize=None)
def _make_sc_kernel(nent, tpw):
    mesh = plsc.VectorSubcoreMesh(core_axis_name="c", subcore_axis_name="s")
    b = NS * tpw
    npair = tpw // (2 * SB)
    w = 6 * SB
    wpad = ((w + 127) // 128) * 128

    @functools.partial(
        pl.kernel,
        mesh=mesh,
        out_type=jax.ShapeDtypeStruct((NC, 2, b), jnp.float32),
        scratch_types=[
            pltpu.VMEM_SHARED((nent,), jnp.float32),
            pltpu.VMEM_SHARED((nent,), jnp.float32),
            pltpu.VMEM((wpad,), jnp.int32),
            pltpu.VMEM((wpad,), jnp.int32),
            pltpu.VMEM((w,), jnp.float32),
            pltpu.VMEM((w,), jnp.float32),
            pltpu.VMEM((tpw,), jnp.float32),
            pltpu.VMEM((tpw,), jnp.float32),
            pltpu.SemaphoreType.DMA,
            pltpu.SemaphoreType.DMA,
            pltpu.SemaphoreType.DMA,
        ],
    )
    def sc_kernel(idx_hbm, ent_t, rel_t, out_hbm,
                  plane_e, plane_r, idx0, idx1, dat0, dat1,
                  psum, nsum, sem_p, sem_i, sem_g):
        ibufs = (idx0, idx1)
        dbufs = (dat0, dat1)

        c = lax.axis_index("c")
        s = lax.axis_index("s")

        def issue_planes(k):
            h0 = None
            for cc in range(NC):
                @pl.when((c == cc) & (s == 0))
                def _():
                    nonlocal h0
                    h = pltpu.async_copy(ent_t.at[cc * DPC + k],
                                         plane_e, sem_p)
                    h0 = h if cc == 0 else h0
                @pl.when((c == cc) & (s == 1))
                def _():
                    pltpu.async_copy(rel_t.at[cc * DPC + k],
                                     plane_r, sem_p)
            return h0

        def wait_planes(handle):
            @pl.when((s == 0) | (s == 1))
            def _():
                handle.wait()

        def issue_idx(set_, sb):
            start = pl.multiple_of(sb * wpad, wpad)
            return pltpu.async_copy(idx_hbm.at[s, pl.ds(start, wpad)],
                                    ibufs[set_], sem_i)

        def drain_idx(set_):
            pltpu.make_async_copy(idx_hbm.at[s, pl.ds(0, wpad)],
                                  ibufs[set_], sem_i).wait()

        def issue_gathers(iset, dset):
            ib = ibufs[iset]
            db = dbufs[dset]
            pltpu.async_copy(
                plane_e.at[ib.at[pl.ds(0, 4 * SB)]],
                db.at[pl.ds(0, 4 * SB)], sem_g)
            pltpu.async_copy(
                plane_r.at[ib.at[pl.ds(4 * SB, 2 * SB)]],
                db.at[pl.ds(4 * SB, 2 * SB)], sem_g)

        def wait_gathers(hs):
            for h in hs:
                h.wait()

        def fold(dset, sb, k):
            db = dbufs[dset]
            base = sb * SB

            def grp(g, _):
                def role(r):
                    return db[pl.ds(
                        pl.multiple_of(r * SB + g * LANES, LANES), LANES)]
                osl = pl.ds(pl.multiple_of(base + g * LANES, LANES), LANES)
                dp = role(0) + role(4) - role(1) + jnp.float32(EPS)
                dn = role(2) + role(5) - role(3) + jnp.float32(EPS)
                if k == 0:
                    psum[osl] = dp * dp
                    nsum[osl] = dn * dn
                else:
                    psum[osl] = psum[osl] + dp * dp
                    nsum[osl] = nsum[osl] + dn * dn
                return 0

            lax.fori_loop(0, SB // LANES, grp, 0)

        def drain_gathers(dset):
            db = dbufs[dset]
            pltpu.make_async_copy(
                plane_e.at[ibufs[0].at[pl.ds(0, 4 * SB)]],
                db.at[pl.ds(0, 4 * SB)], sem_g).wait()
            pltpu.make_async_copy(
                plane_r.at[ibufs[0].at[pl.ds(4 * SB, 2 * SB)]],
                db.at[pl.ds(4 * SB, 2 * SB)], sem_g).wait()

        hplane = issue_planes(0)
        issue_idx(0, 0)
        issue_idx(1, 1)

        for k in range(DPC):
            wait_planes(hplane)
            plsc.subcore_barrier()

            drain_idx(0)
            issue_gathers(0, 0)

            def pair(j, _):
                drain_idx(1)
                issue_gathers(1, 1)
                issue_idx(0, 2 * j + 2)
                drain_gathers(0)
                fold(0, 2 * j, k)
                drain_idx(0)
                issue_gathers(0, 0)
                issue_idx(1, 2 * j + 3)
                drain_gathers(1)
                fold(1, 2 * j + 1, k)
                return 0

            lax.fori_loop(0, npair - 1, pair, 0)

            drain_idx(1)
            issue_gathers(1, 1)
            issue_idx(0, 0)
            drain_gathers(0)
            fold(0, 2 * npair - 2, k)
            issue_idx(1, 1)
            drain_gathers(1)
            fold(1, 2 * npair - 1, k)

            plsc.subcore_barrier()
            if k + 1 < DPC:
                hplane = issue_planes(k + 1)

        drain_idx(0)
        drain_idx(1)

        pltpu.sync_copy(psum, out_hbm.at[c, 0, pl.ds(s * tpw, tpw)])
        pltpu.sync_copy(nsum, out_hbm.at[c, 1, pl.ds(s * tpw, tpw)])

    return sc_kernel


def _finish_body(inv_b, parts_ref, o_ref):
    p = parts_ref[0, 0, :] + parts_ref[1, 0, :]
    n = parts_ref[0, 1, :] + parts_ref[1, 1, :]
    hinge = jnp.maximum(jnp.sqrt(p) - jnp.sqrt(n) + jnp.float32(MARGIN),
                        jnp.float32(0.0))
    o_ref[0, 0] = jnp.sum(hinge) * jnp.float32(inv_b)


@functools.lru_cache(maxsize=None)
def _make_finish(b):
    return pl.pallas_call(
        functools.partial(_finish_body, 1.0 / b),
        out_shape=jax.ShapeDtypeStruct((1, 1), jnp.float32),
        out_specs=pl.BlockSpec(memory_space=pltpu.SMEM),
    )


def kernel(posX, negX, alpha, beta, entityEmbed, relationEmbed):
    b = posX.shape[0]
    tpw = b // NS
    nsub = tpw // SB
    nent = entityEmbed.shape[0]
    w = 6 * SB
    wpad = ((w + 127) // 128) * 128
    x = jnp.concatenate([posX, negX], axis=1)
    xp = x[:, jnp.array([0, 2, 3, 5, 1, 4])]
    idx = (xp.reshape(NS, nsub, SB, 6)
             .transpose(0, 1, 3, 2)
             .reshape(NS, nsub, w))
    idx = jnp.pad(idx, ((0, 0), (0, 0), (0, wpad - w)))
    idx = idx.reshape(NS, nsub * wpad)
    parts = _make_sc_kernel(nent, tpw)(
        idx, entityEmbed.T, relationEmbed.T)
    return _make_finish(b)(parts)[0, 0]
